# Initial kernel scaffold; baseline (speedup 1.0000x reference)
import jax
import jax.numpy as jnp
from jax.experimental import pallas as pl


def kernel(query, top_k, base_embeddings, overlay, confidence, Wq, bq, Wk, bk, Wv, bv, Wo1, bo1, Wo2, bo2, gamma, beta):
    raise NotImplementedError("write your pallas kernel here")


# speedup vs baseline: 10.0896x; 10.0896x over previous
import functools
import math

import jax
import jax.numpy as jnp
from jax.experimental import pallas as pl

K_STATIC = 64
GK = 8
LG = 128
NEG = -1.0e30
CONF_PAD = -5.0e29


def _proj_kernel(base_ref, ovl_ref, conf_ref, wkt_ref, bk_ref, wvt_ref, bv_ref,
                 k_ref, v_ref, bias_ref):
    emb = base_ref[...] + conf_ref[...] * ovl_ref[...]
    k_ref[...] = jnp.dot(emb, wkt_ref[...],
                         preferred_element_type=jnp.float32) + bk_ref[...]
    v_ref[...] = jnp.dot(emb, wvt_ref[...],
                         preferred_element_type=jnp.float32) + bv_ref[...]
    bias_ref[...] = conf_ref[...] * 2.0


def _attn_kernel(q_ref, wqt_ref, bq_ref, k_ref, v_ref, bias_ref,
                 wo1t_ref, bo1_ref, wo2t_ref, bo2_ref, gamma_ref, beta_ref,
                 out_ref, sw_ref, *, R, VP, NG, D):
    inv_sqrt_d = 1.0 / math.sqrt(D)
    query = q_ref[...]
    q = jnp.dot(query, wqt_ref[...],
                preferred_element_type=jnp.float32) + bq_ref[...]
    s = jax.lax.dot_general(q, k_ref[...], (((1,), (1,)), ((), ())),
                            preferred_element_type=jnp.float32)
    s = s * inv_sqrt_d + bias_ref[...]

    s3 = s.reshape(R, NG, LG)
    prev = jnp.full((R, NG, 1), 3.0e38, dtype=jnp.float32)
    cands = []
    for _ in range(GK):
        cur = jnp.max(jnp.where(s3 < prev, s3, NEG), axis=-1, keepdims=True)
        cands.append(cur)
        prev = cur
    cand = jnp.concatenate(cands, axis=-1).reshape(R, NG * GK)

    ncand = NG * GK
    lane = jax.lax.broadcasted_iota(jnp.int32, (R, ncand), 1)
    big_i = jnp.int32(2 ** 30)
    svals = []
    for _ in range(K_STATIC):
        m = jnp.max(cand, axis=-1, keepdims=True)
        svals.append(m)
        eq = cand == m
        first = jnp.min(jnp.where(eq, lane, big_i), axis=-1, keepdims=True)
        cand = jnp.where(lane == first, NEG, cand)
    sv = jnp.concatenate(svals, axis=-1)

    m_row = sv[:, 0:1]
    thresh = sv[:, K_STATIC - 1:K_STATIC]
    e = jnp.exp(sv - m_row)
    denom = jnp.sum(e, axis=-1, keepdims=True)
    sw_ref[...] = e / denom

    w = jnp.where(s >= thresh, jnp.exp(s - m_row), 0.0)
    r = jnp.dot(w, v_ref[...], preferred_element_type=jnp.float32) / denom

    h = jnp.dot(r, wo1t_ref[...],
                preferred_element_type=jnp.float32) + bo1_ref[...]
    h = 0.5 * h * (1.0 + jax.lax.erf(h * (1.0 / math.sqrt(2.0))))
    o = jnp.dot(h, wo2t_ref[...],
                preferred_element_type=jnp.float32) + bo2_ref[...]

    x = query + o
    mu = jnp.mean(x, axis=-1, keepdims=True)
    var = jnp.mean((x - mu) ** 2, axis=-1, keepdims=True)
    out_ref[...] = ((x - mu) / jnp.sqrt(var + 1e-5)) * gamma_ref[...] \
        + beta_ref[...]


def kernel(query, top_k, base_embeddings, overlay, confidence, Wq, bq, Wk, bk,
           Wv, bv, Wo1, bo1, Wo2, bo2, gamma, beta):
    del top_k
    B, S, D = query.shape
    V = base_embeddings.shape[0]
    VP = ((V + LG - 1) // LG) * LG
    NG = VP // LG
    R = 64 if S % 64 == 0 else S
    VB = 1024 if VP % 1024 == 0 else VP

    query2 = query.reshape(S, D)
    pad = VP - V
    base_p = jnp.pad(base_embeddings, ((0, pad), (0, 0)))
    ovl_p = jnp.pad(overlay, ((0, pad), (0, 0)))
    conf_p = jnp.pad(confidence, (0, pad),
                     constant_values=CONF_PAD).reshape(VP, 1)

    wqt, wkt, wvt = Wq.T, Wk.T, Wv.T
    wo1t, wo2t = Wo1.T, Wo2.T
    bq2, bk2, bv2 = bq.reshape(1, D), bk.reshape(1, D), bv.reshape(1, D)
    bo12, bo22 = bo1.reshape(1, D), bo2.reshape(1, D)
    gamma2, beta2 = gamma.reshape(1, D), beta.reshape(1, D)

    const_spec = pl.BlockSpec((D, D), lambda i: (0, 0))
    row_spec = pl.BlockSpec((1, D), lambda i: (0, 0))
    kf, vf, biasf = pl.pallas_call(
        _proj_kernel,
        grid=(VP // VB,),
        in_specs=[
            pl.BlockSpec((VB, D), lambda i: (i, 0)),
            pl.BlockSpec((VB, D), lambda i: (i, 0)),
            pl.BlockSpec((VB, 1), lambda i: (i, 0)),
            const_spec, row_spec, const_spec, row_spec,
        ],
        out_specs=[
            pl.BlockSpec((VB, D), lambda i: (i, 0)),
            pl.BlockSpec((VB, D), lambda i: (i, 0)),
            pl.BlockSpec((VB, 1), lambda i: (i, 0)),
        ],
        out_shape=[
            jax.ShapeDtypeStruct((VP, D), jnp.float32),
            jax.ShapeDtypeStruct((VP, D), jnp.float32),
            jax.ShapeDtypeStruct((VP, 1), jnp.float32),
        ],
    )(base_p, ovl_p, conf_p, wkt, bk2, wvt, bv2)

    bias_row = biasf.reshape(1, VP)

    attn = functools.partial(_attn_kernel, R=R, VP=VP, NG=NG, D=D)
    out, sw = pl.pallas_call(
        attn,
        grid=(S // R,),
        in_specs=[
            pl.BlockSpec((R, D), lambda i: (i, 0)),
            const_spec, row_spec,
            pl.BlockSpec((VP, D), lambda i: (0, 0)),
            pl.BlockSpec((VP, D), lambda i: (0, 0)),
            pl.BlockSpec((1, VP), lambda i: (0, 0)),
            const_spec, row_spec, const_spec, row_spec, row_spec, row_spec,
        ],
        out_specs=[
            pl.BlockSpec((R, D), lambda i: (i, 0)),
            pl.BlockSpec((R, K_STATIC), lambda i: (i, 0)),
        ],
        out_shape=[
            jax.ShapeDtypeStruct((S, D), jnp.float32),
            jax.ShapeDtypeStruct((S, K_STATIC), jnp.float32),
        ],
    )(query2, wqt, bq2, kf, vf, bias_row,
      wo1t, bo12, wo2t, bo22, gamma2, beta2)

    return out.reshape(B, S, D), sw.reshape(B, S, K_STATIC)


# speedup vs baseline: 24.5395x; 2.4322x over previous
import functools
import math

import jax
import jax.numpy as jnp
from jax.experimental import pallas as pl

K_STATIC = 64
GK = 6
LG = 128
NEG = -1.0e30
CONF_PAD = -5.0e29


def _proj_kernel(base_ref, ovl_ref, conf_ref, wkt_ref, bk_ref, wvt_ref, bv_ref,
                 k_ref, v_ref, bias_ref):
    emb = base_ref[...] + conf_ref[...] * ovl_ref[...]
    k_ref[...] = jnp.dot(emb, wkt_ref[...],
                         preferred_element_type=jnp.float32) + bk_ref[...]
    v_ref[...] = jnp.dot(emb, wvt_ref[...],
                         preferred_element_type=jnp.float32) + bv_ref[...]
    bias_ref[...] = conf_ref[...] * 2.0


def _attn_kernel(q_ref, wqt_ref, bq_ref, k_ref, v_ref, bias_ref,
                 wo1t_ref, bo1_ref, wo2t_ref, bo2_ref, gamma_ref, beta_ref,
                 out_ref, sw_ref, *, R, VP, NG, D):
    inv_sqrt_d = 1.0 / math.sqrt(D)
    query = q_ref[...]
    q = jnp.dot(query, wqt_ref[...],
                preferred_element_type=jnp.float32) + bq_ref[...]
    s = jax.lax.dot_general(q, k_ref[...], (((1,), (1,)), ((), ())),
                            preferred_element_type=jnp.float32)
    s = s * inv_sqrt_d + bias_ref[...]

    s3 = s.reshape(R, NG, LG)
    prev = jnp.full((R, 1, LG), 3.0e38, dtype=jnp.float32)
    cands = []
    for _ in range(GK):
        cur = jnp.max(jnp.where(s3 < prev, s3, NEG), axis=1, keepdims=True)
        cands.append(cur.reshape(R, LG))
        prev = cur
    cand = jnp.concatenate(cands, axis=-1)

    svals, scnts = [], []
    for _ in range(K_STATIC):
        m = jnp.max(cand, axis=-1, keepdims=True)
        eq = cand == m
        svals.append(m)
        scnts.append(jnp.sum(eq.astype(jnp.float32), axis=-1, keepdims=True))
        cand = jnp.where(eq, NEG, cand)
    sv0 = jnp.concatenate(svals, axis=-1)
    cnt = jnp.concatenate(scnts, axis=-1)
    ends = cnt
    for d in (1, 2, 4, 8, 16, 32):
        ends = ends + jnp.pad(ends[:, :-d], ((0, 0), (d, 0)))
    starts = ends - cnt
    slot = jax.lax.broadcasted_iota(
        jnp.int32, (1, 1, K_STATIC), 2).astype(jnp.float32)
    win = (starts[:, :, None] <= slot) & (ends[:, :, None] > slot)
    sv = jnp.sum(jnp.where(win, sv0[:, :, None], 0.0), axis=1)

    m_row = sv[:, 0:1]
    thresh = sv[:, K_STATIC - 1:K_STATIC]
    e = jnp.exp(sv - m_row)
    denom = jnp.sum(e, axis=-1, keepdims=True)
    sw_ref[...] = e / denom

    w = jnp.where(s >= thresh, jnp.exp(s - m_row), 0.0)
    r = jnp.dot(w, v_ref[...], preferred_element_type=jnp.float32) / denom

    h = jnp.dot(r, wo1t_ref[...],
                preferred_element_type=jnp.float32) + bo1_ref[...]
    h = 0.5 * h * (1.0 + jax.lax.erf(h * (1.0 / math.sqrt(2.0))))
    o = jnp.dot(h, wo2t_ref[...],
                preferred_element_type=jnp.float32) + bo2_ref[...]

    x = query + o
    mu = jnp.mean(x, axis=-1, keepdims=True)
    var = jnp.mean((x - mu) ** 2, axis=-1, keepdims=True)
    out_ref[...] = ((x - mu) / jnp.sqrt(var + 1e-5)) * gamma_ref[...] \
        + beta_ref[...]


def kernel(query, top_k, base_embeddings, overlay, confidence, Wq, bq, Wk, bk,
           Wv, bv, Wo1, bo1, Wo2, bo2, gamma, beta):
    del top_k
    B, S, D = query.shape
    V = base_embeddings.shape[0]
    VP = ((V + LG - 1) // LG) * LG
    NG = VP // LG
    R = 128 if S % 128 == 0 else S
    VB = 1024 if VP % 1024 == 0 else VP

    query2 = query.reshape(S, D)
    pad = VP - V
    base_p = jnp.pad(base_embeddings, ((0, pad), (0, 0)))
    ovl_p = jnp.pad(overlay, ((0, pad), (0, 0)))
    conf_p = jnp.pad(confidence, (0, pad),
                     constant_values=CONF_PAD).reshape(VP, 1)

    wqt, wkt, wvt = Wq.T, Wk.T, Wv.T
    wo1t, wo2t = Wo1.T, Wo2.T
    bq2, bk2, bv2 = bq.reshape(1, D), bk.reshape(1, D), bv.reshape(1, D)
    bo12, bo22 = bo1.reshape(1, D), bo2.reshape(1, D)
    gamma2, beta2 = gamma.reshape(1, D), beta.reshape(1, D)

    const_spec = pl.BlockSpec((D, D), lambda i: (0, 0))
    row_spec = pl.BlockSpec((1, D), lambda i: (0, 0))
    kf, vf, biasf = pl.pallas_call(
        _proj_kernel,
        grid=(VP // VB,),
        in_specs=[
            pl.BlockSpec((VB, D), lambda i: (i, 0)),
            pl.BlockSpec((VB, D), lambda i: (i, 0)),
            pl.BlockSpec((VB, 1), lambda i: (i, 0)),
            const_spec, row_spec, const_spec, row_spec,
        ],
        out_specs=[
            pl.BlockSpec((VB, D), lambda i: (i, 0)),
            pl.BlockSpec((VB, D), lambda i: (i, 0)),
            pl.BlockSpec((VB, 1), lambda i: (i, 0)),
        ],
        out_shape=[
            jax.ShapeDtypeStruct((VP, D), jnp.float32),
            jax.ShapeDtypeStruct((VP, D), jnp.float32),
            jax.ShapeDtypeStruct((VP, 1), jnp.float32),
        ],
    )(base_p, ovl_p, conf_p, wkt, bk2, wvt, bv2)

    bias_row = biasf.reshape(1, VP)

    attn = functools.partial(_attn_kernel, R=R, VP=VP, NG=NG, D=D)
    out, sw = pl.pallas_call(
        attn,
        grid=(S // R,),
        in_specs=[
            pl.BlockSpec((R, D), lambda i: (i, 0)),
            const_spec, row_spec,
            pl.BlockSpec((VP, D), lambda i: (0, 0)),
            pl.BlockSpec((VP, D), lambda i: (0, 0)),
            pl.BlockSpec((1, VP), lambda i: (0, 0)),
            const_spec, row_spec, const_spec, row_spec, row_spec, row_spec,
        ],
        out_specs=[
            pl.BlockSpec((R, D), lambda i: (i, 0)),
            pl.BlockSpec((R, K_STATIC), lambda i: (i, 0)),
        ],
        out_shape=[
            jax.ShapeDtypeStruct((S, D), jnp.float32),
            jax.ShapeDtypeStruct((S, K_STATIC), jnp.float32),
        ],
    )(query2, wqt, bq2, kf, vf, bias_row,
      wo1t, bo12, wo2t, bo22, gamma2, beta2)

    return out.reshape(B, S, D), sw.reshape(B, S, K_STATIC)
